# baseline (device time: 34075 ns/iter reference)
import jax
import jax.numpy as jnp
from jax import lax
from jax.experimental import pallas as pl
from jax.experimental.pallas import tpu as pltpu

N_DEV = 16
BM = 256


def kernel(x, dy, gamma):
    m, d = x.shape
    n_blocks = m // BM

    def body(x_ref, dy_ref, out_ref, acc_ref, comm_ref, send_sems, recv_sems):
        my = lax.axis_index("i")
        step = pl.program_id(0)

        xv = x_ref[:, :]
        dyv = dy_ref[:, :]
        ones_d = jnp.ones((d, 1), jnp.float32)
        s1 = jnp.dot(xv, ones_d, preferred_element_type=jnp.float32)
        s2 = jnp.dot(xv * xv, ones_d, preferred_element_type=jnp.float32)
        mu = s1 * (1.0 / d)
        var = s2 * (1.0 / d) - mu * mu
        rstd = lax.rsqrt(var + 1e-5)
        p = xv * dyv
        w = jnp.concatenate(
            [rstd, rstd * mu, jnp.ones((BM, 1), jnp.float32)], axis=1
        )
        g1 = lax.dot_general(
            w[:, 0:1], p, (((0,), (0,)), ((), ())),
            preferred_element_type=jnp.float32,
        )
        g2 = lax.dot_general(
            w[:, 1:3], dyv, (((0,), (0,)), ((), ())),
            preferred_element_type=jnp.float32,
        )
        partial = jnp.concatenate([g1 - g2[0:1], g2[1:2]], axis=0)

        @pl.when(step == 0)
        def _():
            acc_ref[:, :] = partial

        @pl.when(step != 0)
        def _():
            acc_ref[:, :] = acc_ref[:, :] + partial

        @pl.when(step == n_blocks - 1)
        def _():
            comm_ref[my] = acc_ref[:, :]
            rdmas = []
            for k in range(1, N_DEV):
                peer = lax.rem(my + k, N_DEV)
                rdma = pltpu.make_async_remote_copy(
                    src_ref=comm_ref.at[my],
                    dst_ref=comm_ref.at[my],
                    send_sem=send_sems.at[k - 1],
                    recv_sem=recv_sems.at[k - 1],
                    device_id=(peer,),
                    device_id_type=pl.DeviceIdType.MESH,
                )
                rdma.start()
                rdmas.append(rdma)
            for rdma in rdmas:
                rdma.wait_send()
            for k in range(1, N_DEV):
                src = lax.rem(my + (N_DEV - k), N_DEV)
                recv = pltpu.make_async_remote_copy(
                    src_ref=comm_ref.at[my],
                    dst_ref=comm_ref.at[src],
                    send_sem=send_sems.at[k - 1],
                    recv_sem=recv_sems.at[k - 1],
                    device_id=(src,),
                    device_id_type=pl.DeviceIdType.MESH,
                )
                recv.wait_recv()
            out_ref[:, :] = jnp.sum(comm_ref[:, :, :], axis=0)

    return pl.pallas_call(
        body,
        grid=(n_blocks,),
        out_shape=jax.ShapeDtypeStruct((2, d), jnp.float32),
        in_specs=[
            pl.BlockSpec((BM, d), lambda i: (i, 0)),
            pl.BlockSpec((BM, d), lambda i: (i, 0)),
        ],
        out_specs=pl.BlockSpec((2, d), lambda i: (0, 0)),
        scratch_shapes=[
            pltpu.VMEM((2, d), jnp.float32),
            pltpu.VMEM((N_DEV, 2, d), jnp.float32),
            pltpu.SemaphoreType.DMA((N_DEV - 1,)),
            pltpu.SemaphoreType.DMA((N_DEV - 1,)),
        ],
    )(x, dy)


# device time: 29211 ns/iter; 1.1665x vs baseline; 1.1665x over previous
import jax
import jax.numpy as jnp
from jax import lax
from jax.experimental import pallas as pl
from jax.experimental.pallas import tpu as pltpu

N_DEV = 16
BM = 256


def kernel(x, dy, gamma):
    m, d = x.shape
    n_blocks = m // BM

    def body(x_ref, dy_ref, out_ref, acc_ref, comm_ref, send_sems, recv_sems):
        my = lax.axis_index("i")
        step = pl.program_id(0)

        xv = x_ref[:, :]
        dyv = dy_ref[:, :]
        dgamma = jnp.sum(xv, axis=0)[None, :]
        dbeta = jnp.sum(dyv, axis=0)[None, :]
        partial = jnp.concatenate([dgamma, dbeta], axis=0)

        @pl.when(step == 0)
        def _():
            acc_ref[:, :] = partial

        @pl.when(step != 0)
        def _():
            acc_ref[:, :] = acc_ref[:, :] + partial

        @pl.when(step == n_blocks - 1)
        def _():
            comm_ref[my] = acc_ref[:, :]
            rdmas = []
            for k in range(1, N_DEV):
                peer = lax.rem(my + k, N_DEV)
                rdma = pltpu.make_async_remote_copy(
                    src_ref=comm_ref.at[my],
                    dst_ref=comm_ref.at[my],
                    send_sem=send_sems.at[k - 1],
                    recv_sem=recv_sems.at[k - 1],
                    device_id=(peer,),
                    device_id_type=pl.DeviceIdType.MESH,
                )
                rdma.start()
                rdmas.append(rdma)
            for rdma in rdmas:
                rdma.wait_send()
            for k in range(1, N_DEV):
                src = lax.rem(my + (N_DEV - k), N_DEV)
                recv = pltpu.make_async_remote_copy(
                    src_ref=comm_ref.at[my],
                    dst_ref=comm_ref.at[src],
                    send_sem=send_sems.at[k - 1],
                    recv_sem=recv_sems.at[k - 1],
                    device_id=(src,),
                    device_id_type=pl.DeviceIdType.MESH,
                )
                recv.wait_recv()
            out_ref[:, :] = jnp.sum(comm_ref[:, :, :], axis=0)

    return pl.pallas_call(
        body,
        grid=(n_blocks,),
        out_shape=jax.ShapeDtypeStruct((2, d), jnp.float32),
        in_specs=[
            pl.BlockSpec((BM, d), lambda i: (i, 0)),
            pl.BlockSpec((BM, d), lambda i: (i, 0)),
        ],
        out_specs=pl.BlockSpec((2, d), lambda i: (0, 0)),
        scratch_shapes=[
            pltpu.VMEM((2, d), jnp.float32),
            pltpu.VMEM((N_DEV, 2, d), jnp.float32),
            pltpu.SemaphoreType.DMA((N_DEV - 1,)),
            pltpu.SemaphoreType.DMA((N_DEV - 1,)),
        ],
    )(x, dy)


# device time: 28646 ns/iter; 1.1895x vs baseline; 1.0197x over previous
import jax
import jax.numpy as jnp
from jax import lax
from jax.experimental import pallas as pl
from jax.experimental.pallas import tpu as pltpu

N_DEV = 16
CHUNK = 256


def kernel(x, dy, gamma):
    m, d = x.shape
    n_chunks = m // CHUNK

    def body(x_hbm, dy_hbm, out_ref, xbuf, dybuf, comm_ref,
             load_sems, send_sems, recv_sems):
        my = lax.axis_index("i")

        def chunk_copy(inp, hbm, buf, c):
            return pltpu.make_async_copy(
                hbm.at[pl.ds(c * CHUNK, CHUNK), :],
                buf.at[pl.ds(c * CHUNK, CHUNK), :],
                load_sems.at[inp, c],
            )

        for c in range(n_chunks):
            chunk_copy(0, x_hbm, xbuf, c).start()
            chunk_copy(1, dy_hbm, dybuf, c).start()

        acc = jnp.zeros((2, d), jnp.float32)
        for c in range(n_chunks):
            chunk_copy(0, x_hbm, xbuf, c).wait()
            chunk_copy(1, dy_hbm, dybuf, c).wait()
            xv = xbuf[pl.ds(c * CHUNK, CHUNK), :]
            dyv = dybuf[pl.ds(c * CHUNK, CHUNK), :]
            mu = jnp.mean(xv, axis=1, keepdims=True)
            xc = xv - mu
            var = jnp.mean(xc * xc, axis=1, keepdims=True)
            rstd = lax.rsqrt(var + 1e-5)
            xhat = xc * rstd
            dgamma = jnp.sum(dyv * xhat, axis=0)[None, :]
            dbeta = jnp.sum(dyv, axis=0)[None, :]
            acc = acc + jnp.concatenate([dgamma, dbeta], axis=0)

        comm_ref[my] = acc

        rdmas = []
        for k in range(1, N_DEV):
            peer = lax.rem(my + k, N_DEV)
            rdma = pltpu.make_async_remote_copy(
                src_ref=comm_ref.at[my],
                dst_ref=comm_ref.at[my],
                send_sem=send_sems.at[k - 1],
                recv_sem=recv_sems.at[k - 1],
                device_id=(peer,),
                device_id_type=pl.DeviceIdType.MESH,
            )
            rdma.start()
            rdmas.append(rdma)
        for rdma in rdmas:
            rdma.wait_send()
        for k in range(1, N_DEV):
            src = lax.rem(my + (N_DEV - k), N_DEV)
            recv = pltpu.make_async_remote_copy(
                src_ref=comm_ref.at[my],
                dst_ref=comm_ref.at[src],
                send_sem=send_sems.at[k - 1],
                recv_sem=recv_sems.at[k - 1],
                device_id=(src,),
                device_id_type=pl.DeviceIdType.MESH,
            )
            recv.wait_recv()
        out_ref[:, :] = jnp.sum(comm_ref[:, :, :], axis=0)

    return pl.pallas_call(
        body,
        out_shape=jax.ShapeDtypeStruct((2, d), jnp.float32),
        in_specs=[
            pl.BlockSpec(memory_space=pl.ANY),
            pl.BlockSpec(memory_space=pl.ANY),
        ],
        out_specs=pl.BlockSpec(memory_space=pltpu.VMEM),
        scratch_shapes=[
            pltpu.VMEM((m, d), jnp.float32),
            pltpu.VMEM((m, d), jnp.float32),
            pltpu.VMEM((N_DEV, 2, d), jnp.float32),
            pltpu.SemaphoreType.DMA((2, m // CHUNK)),
            pltpu.SemaphoreType.DMA((N_DEV - 1,)),
            pltpu.SemaphoreType.DMA((N_DEV - 1,)),
        ],
        compiler_params=pltpu.CompilerParams(vmem_limit_bytes=60 * 1024 * 1024),
    )(x, dy)


# device time: 28427 ns/iter; 1.1987x vs baseline; 1.0077x over previous
import jax
import jax.numpy as jnp
from jax import lax
from jax.experimental import pallas as pl
from jax.experimental.pallas import tpu as pltpu

N_DEV = 16
CHUNK = 128


def kernel(x, dy, gamma):
    m, d = x.shape
    n_chunks = m // CHUNK

    def body(x_hbm, dy_hbm, out_ref, xbuf, dybuf, comm_ref,
             load_sems, send_sems, recv_sems):
        my = lax.axis_index("i")

        def chunk_copy(inp, hbm, buf, c):
            return pltpu.make_async_copy(
                hbm.at[pl.ds(c * CHUNK, CHUNK), :],
                buf.at[pl.ds(c * CHUNK, CHUNK), :],
                load_sems.at[inp, c],
            )

        for c in range(n_chunks):
            chunk_copy(0, x_hbm, xbuf, c).start()
            chunk_copy(1, dy_hbm, dybuf, c).start()

        acc = jnp.zeros((2, d), jnp.float32)
        for c in range(n_chunks):
            chunk_copy(0, x_hbm, xbuf, c).wait()
            chunk_copy(1, dy_hbm, dybuf, c).wait()
            xv = xbuf[pl.ds(c * CHUNK, CHUNK), :]
            dyv = dybuf[pl.ds(c * CHUNK, CHUNK), :]
            mu = jnp.mean(xv, axis=1, keepdims=True)
            xc = xv - mu
            var = jnp.mean(xc * xc, axis=1, keepdims=True)
            rstd = lax.rsqrt(var + 1e-5)
            xhat = xc * rstd
            dgamma = jnp.sum(dyv * xhat, axis=0)[None, :]
            dbeta = jnp.sum(dyv, axis=0)[None, :]
            acc = acc + jnp.concatenate([dgamma, dbeta], axis=0)

        comm_ref[my] = acc

        rdmas = []
        for k in range(1, N_DEV):
            peer = lax.rem(my + k, N_DEV)
            rdma = pltpu.make_async_remote_copy(
                src_ref=comm_ref.at[my],
                dst_ref=comm_ref.at[my],
                send_sem=send_sems.at[k - 1],
                recv_sem=recv_sems.at[k - 1],
                device_id=(peer,),
                device_id_type=pl.DeviceIdType.MESH,
            )
            rdma.start()
            rdmas.append(rdma)
        for rdma in rdmas:
            rdma.wait_send()
        for k in range(1, N_DEV):
            src = lax.rem(my + (N_DEV - k), N_DEV)
            recv = pltpu.make_async_remote_copy(
                src_ref=comm_ref.at[my],
                dst_ref=comm_ref.at[src],
                send_sem=send_sems.at[k - 1],
                recv_sem=recv_sems.at[k - 1],
                device_id=(src,),
                device_id_type=pl.DeviceIdType.MESH,
            )
            recv.wait_recv()
        out_ref[:, :] = jnp.sum(comm_ref[:, :, :], axis=0)

    return pl.pallas_call(
        body,
        out_shape=jax.ShapeDtypeStruct((2, d), jnp.float32),
        in_specs=[
            pl.BlockSpec(memory_space=pl.ANY),
            pl.BlockSpec(memory_space=pl.ANY),
        ],
        out_specs=pl.BlockSpec(memory_space=pltpu.VMEM),
        scratch_shapes=[
            pltpu.VMEM((m, d), jnp.float32),
            pltpu.VMEM((m, d), jnp.float32),
            pltpu.VMEM((N_DEV, 2, d), jnp.float32),
            pltpu.SemaphoreType.DMA((2, m // CHUNK)),
            pltpu.SemaphoreType.DMA((N_DEV - 1,)),
            pltpu.SemaphoreType.DMA((N_DEV - 1,)),
        ],
        compiler_params=pltpu.CompilerParams(vmem_limit_bytes=60 * 1024 * 1024),
    )(x, dy)
